# baseline (speedup 1.0000x reference)
import functools

import jax
import jax.numpy as jnp
from jax import lax
from jax.experimental import pallas as pl
from jax.experimental.pallas import tpu as pltpu
from jax.experimental.pallas import tpu_sc as plsc

M = 100000
D = 128
B = 16384

NC = 2
NS = 16
L = 16

RB = 32
S = 12800
SBLK = S // RB
SWEEPS = 4
FULL = 3
BPW = SBLK // NS
SPW = BPW * RB
HPW = SPW // 2
EPW = B // NS
CH = 32
NCHUNK = EPW // CH
NV = CH // L
NBUF = 5
GR = 4


def _body(x_hbm, idx_hbm, src_hbm, alpha_hbm, out_hbm,
          idx_v, tgt_v, alpha_v, slab, *rest):
    bufs = rest[:NBUF]
    lsems = rest[NBUF:2 * NBUF]
    asems = rest[2 * NBUF:3 * NBUF]
    hsem0, hsem1, ssem0, ssem1 = rest[3 * NBUF:]
    c = lax.axis_index("c")
    s = lax.axis_index("s")
    lane = lax.iota(jnp.int32, L)
    garb = S + s * GR + jnp.bitwise_and(lane, GR - 1)

    def base_of(t):
        return (NC * t + c) * S

    def halves_of(t):
        b = base_of(t) + s * SPW
        l0 = s * SPW
        return ((b, l0), (b + HPW, l0 + HPW))

    def load_half(t, h, sem):
        (g, l) = halves_of(t)[h]
        return pltpu.async_copy(x_hbm.at[pl.ds(g, HPW)],
                                slab.at[pl.ds(l, HPW)], sem)

    def store_half(t, h, sem):
        (g, l) = halves_of(t)[h]
        return pltpu.async_copy(slab.at[pl.ds(l, HPW)],
                                out_hbm.at[pl.ds(g, HPW)], sem)

    def partial_copy(t, to_out):
        base_row = base_of(t)
        rows = jnp.clip(M - base_row, 0, S)
        my_b0 = s * BPW
        my_n = jnp.clip(rows // RB - my_b0, 0, BPW)

        def _copy(loc, glob, n):
            if to_out:
                pltpu.sync_copy(slab.at[pl.ds(loc, n)],
                                out_hbm.at[pl.ds(glob, n)])
            else:
                pltpu.sync_copy(x_hbm.at[pl.ds(glob, n)],
                                slab.at[pl.ds(loc, n)])

        @pl.when(my_n == BPW)
        def _full():
            _copy(my_b0 * RB, base_row + my_b0 * RB, SPW)

        @pl.when(jnp.logical_and(my_n > 0, my_n < BPW))
        def _part():
            def blk(j, _):
                o = (my_b0 + j) * RB
                _copy(o, base_row + o, RB)
                return 0
            lax.fori_loop(0, my_n, blk, 0)

    ld = (load_half(0, 0, hsem0), load_half(0, 1, hsem1))
    pltpu.sync_copy(idx_hbm.at[pl.ds(s * EPW, EPW)], idx_v)
    pltpu.sync_copy(alpha_hbm, alpha_v)
    av = alpha_v[pl.ds(0, L)]
    do_scale = av[0] != 1.0

    for t in range(SWEEPS):
        base_row = base_of(t)
        rows = jnp.clip(M - base_row, 0, S)

        def _load(j):
            return pltpu.async_copy(
                src_hbm.at[pl.ds(s * EPW + j * CH, CH)],
                bufs[j % NBUF], lsems[j % NBUF])

        loads = {0: _load(0), 1: _load(1)}
        lo = base_row
        hi = base_row + rows
        for r in range(NCHUNK):
            for q in range(NV):
                vec = idx_v[pl.ds((r * NV + q) * L, L)]
                in_slab = (vec >= lo) & (vec < hi)
                tgt_v[r, pl.ds(q * L, L)] = jnp.where(in_slab, vec - lo, garb)

        if ld is not None:
            ld[0].wait()
            ld[1].wait()
            ld = None
        plsc.subcore_barrier()

        adds = {}
        for j in range(NCHUNK):
            loads[j].wait()
            bj = bufs[j % NBUF]

            @pl.when(do_scale)
            def _scale(bj=bj):
                def scale_row(rr, _):
                    for q in range(D // L):
                        sl = pl.ds(q * L, L)
                        bj[rr, sl] = bj[rr, sl] * av
                    return 0
                lax.fori_loop(0, CH, scale_row, 0)

            adds[j] = pltpu.async_copy(
                bj, slab.at[tgt_v.at[j]], asems[j % NBUF], add=True)
            if j + 2 < NCHUNK:
                if j - (NBUF - 2) >= 0:
                    adds[j - (NBUF - 2)].wait()
                loads[j + 2] = _load(j + 2)
        for j in range(max(0, NCHUNK - NBUF), NCHUNK):
            adds[j].wait()

        plsc.subcore_barrier()

        if t < FULL:
            st = (store_half(t, 0, ssem0), store_half(t, 1, ssem1))
            if t + 1 < FULL:
                st[0].wait()
                nl0 = load_half(t + 1, 0, hsem0)
                st[1].wait()
                nl1 = load_half(t + 1, 1, hsem1)
                ld = (nl0, nl1)
            else:
                st[0].wait()
                st[1].wait()
                partial_copy(t + 1, to_out=False)
        else:
            partial_copy(t, to_out=True)


_scatter_add = functools.partial(
    pl.kernel,
    mesh=plsc.VectorSubcoreMesh(core_axis_name="c", subcore_axis_name="s"),
    out_type=jax.ShapeDtypeStruct((M, D), jnp.float32),
    scratch_types=(
        [
            pltpu.VMEM((EPW,), jnp.int32),
            pltpu.VMEM((NCHUNK, CH), jnp.int32),
            pltpu.VMEM((L,), jnp.float32),
            pltpu.VMEM_SHARED((S + NS * GR, D), jnp.float32),
        ]
        + [pltpu.VMEM((CH, D), jnp.float32) for _ in range(NBUF)]
        + [pltpu.SemaphoreType.DMA for _ in range(2 * NBUF + 4)]
    ),
)(_body)


def kernel(x, dim, index, source, alpha):
    del dim
    alpha_vec = jnp.full((L,), alpha, dtype=jnp.float32)
    return _scatter_add(x, index, source, alpha_vec)


# speedup vs baseline: 1.1994x; 1.1994x over previous
import functools

import jax
import jax.numpy as jnp
from jax import lax
from jax.experimental import pallas as pl
from jax.experimental.pallas import tpu as pltpu
from jax.experimental.pallas import tpu_sc as plsc

M = 100000
D = 128
B = 16384

NC = 2
NS = 16
L = 16

RB = 32
S = 12800
SBLK = S // RB
SWEEPS = 4
FULL = 3
BPW = SBLK // NS
SPW = BPW * RB
HPW = SPW // 2
EPW = B // NS
CH = 64
NCHUNK = EPW // CH
NV = CH // L
NBUF = 3
GR = 4


def _body(x_hbm, idx_hbm, src_hbm, alpha_hbm, out_hbm,
          idx_v, tgt_v, alpha_v, slab, *rest):
    bufs = rest[:NBUF]
    lsems = rest[NBUF:2 * NBUF]
    asems = rest[2 * NBUF:3 * NBUF]
    hsem0, hsem1, ssem0, ssem1 = rest[3 * NBUF:]
    c = lax.axis_index("c")
    s = lax.axis_index("s")
    lane = lax.iota(jnp.int32, L)
    garb = S + s * GR + jnp.bitwise_and(lane, GR - 1)

    def base_of(t):
        return (NC * t + c) * S

    def halves_of(t):
        g = jnp.minimum(base_of(t) + s * SPW, M - SPW)
        l0 = s * SPW
        return ((g, l0), (g + HPW, l0 + HPW))

    def load_half(t, h, sem):
        (g, l) = halves_of(t)[h]
        return pltpu.async_copy(x_hbm.at[pl.ds(g, HPW)],
                                slab.at[pl.ds(l, HPW)], sem)

    def store_half(t, h, sem):
        (g, l) = halves_of(t)[h]
        return pltpu.async_copy(slab.at[pl.ds(l, HPW)],
                                out_hbm.at[pl.ds(g, HPW)], sem)

    def partial_store(t):
        base_row = base_of(t)
        rows = jnp.clip(M - base_row, 0, S)
        my_b0 = s * BPW
        my_n = jnp.clip(rows // RB - my_b0, 0, BPW)

        @pl.when(my_n == BPW)
        def _full():
            o = my_b0 * RB
            pltpu.sync_copy(slab.at[pl.ds(o, SPW)],
                            out_hbm.at[pl.ds(base_row + o, SPW)])

        @pl.when(jnp.logical_and(my_n > 0, my_n < BPW))
        def _part():
            def blk(j, _):
                o = (my_b0 + j) * RB
                pltpu.sync_copy(slab.at[pl.ds(o, RB)],
                                out_hbm.at[pl.ds(base_row + o, RB)])
                return 0
            lax.fori_loop(0, my_n, blk, 0)

    ld = (load_half(0, 0, hsem0), load_half(0, 1, hsem1))
    pltpu.sync_copy(idx_hbm.at[pl.ds(s * EPW, EPW)], idx_v)
    pltpu.sync_copy(alpha_hbm, alpha_v)
    av = alpha_v[pl.ds(0, L)]
    do_scale = av[0] != 1.0

    for t in range(SWEEPS):
        base_row = base_of(t)
        rows = jnp.clip(M - base_row, 0, S)

        def _load(j):
            return pltpu.async_copy(
                src_hbm.at[pl.ds(s * EPW + j * CH, CH)],
                bufs[j % NBUF], lsems[j % NBUF])

        loads = {0: _load(0), 1: _load(1)}
        lo = base_row
        hi = base_row + rows
        for r in range(NCHUNK):
            for q in range(NV):
                vec = idx_v[pl.ds((r * NV + q) * L, L)]
                in_slab = (vec >= lo) & (vec < hi)
                tgt_v[r, pl.ds(q * L, L)] = jnp.where(in_slab, vec - lo, garb)

        if ld is not None:
            ld[0].wait()
            ld[1].wait()
            ld = None
        plsc.subcore_barrier()

        adds = {}
        for j in range(NCHUNK):
            loads[j].wait()
            bj = bufs[j % NBUF]

            @pl.when(do_scale)
            def _scale(bj=bj):
                def scale_row(rr, _):
                    for q in range(D // L):
                        sl = pl.ds(q * L, L)
                        bj[rr, sl] = bj[rr, sl] * av
                    return 0
                lax.fori_loop(0, CH, scale_row, 0)

            adds[j] = pltpu.async_copy(
                bj, slab.at[tgt_v.at[j]], asems[j % NBUF], add=True)
            if j + 2 < NCHUNK:
                if j - (NBUF - 2) >= 0:
                    adds[j - (NBUF - 2)].wait()
                loads[j + 2] = _load(j + 2)
        for j in range(max(0, NCHUNK - NBUF), NCHUNK):
            adds[j].wait()

        plsc.subcore_barrier()

        if t < FULL:
            st = (store_half(t, 0, ssem0), store_half(t, 1, ssem1))
            st[0].wait()
            nl0 = load_half(t + 1, 0, hsem0)
            st[1].wait()
            nl1 = load_half(t + 1, 1, hsem1)
            ld = (nl0, nl1)
        else:
            partial_store(t)


_scatter_add = functools.partial(
    pl.kernel,
    mesh=plsc.VectorSubcoreMesh(core_axis_name="c", subcore_axis_name="s"),
    out_type=jax.ShapeDtypeStruct((M, D), jnp.float32),
    scratch_types=(
        [
            pltpu.VMEM((EPW,), jnp.int32),
            pltpu.VMEM((NCHUNK, CH), jnp.int32),
            pltpu.VMEM((L,), jnp.float32),
            pltpu.VMEM_SHARED((S + NS * GR, D), jnp.float32),
        ]
        + [pltpu.VMEM((CH, D), jnp.float32) for _ in range(NBUF)]
        + [pltpu.SemaphoreType.DMA for _ in range(2 * NBUF + 4)]
    ),
)(_body)


def kernel(x, dim, index, source, alpha):
    del dim
    alpha_vec = jnp.full((L,), alpha, dtype=jnp.float32)
    return _scatter_add(x, index, source, alpha_vec)


# speedup vs baseline: 1.2089x; 1.0079x over previous
import functools

import jax
import jax.numpy as jnp
from jax import lax
from jax.experimental import pallas as pl
from jax.experimental.pallas import tpu as pltpu
from jax.experimental.pallas import tpu_sc as plsc

M = 100000
D = 128
B = 16384

NC = 2
NS = 16
L = 16

RB = 32
S = 12800
SBLK = S // RB
SWEEPS = 4
FULL = 3
BPW = SBLK // NS
SPW = BPW * RB
HPW = SPW // 2
EPW = B // NS
CH = 64
NCHUNK = EPW // CH
NV = CH // L
NBUF = 3
GR = 4


def _body(x_hbm, idx_hbm, src_hbm, alpha_hbm, out_hbm,
          idx_v, tgt_v, alpha_v, slab, *rest):
    bufs = rest[:NBUF]
    lsems = rest[NBUF:2 * NBUF]
    asems = rest[2 * NBUF:3 * NBUF]
    hsem0, hsem1, ssem0, ssem1 = rest[3 * NBUF:]
    c = lax.axis_index("c")
    s = lax.axis_index("s")
    lane = lax.iota(jnp.int32, L)
    garb = S + s * GR + jnp.bitwise_and(lane, GR - 1)

    def base_of(t):
        return (NC * t + c) * S

    def halves_of(t):
        g = jnp.minimum(base_of(t) + s * SPW, M - SPW)
        l0 = s * SPW
        return ((g, l0), (g + HPW, l0 + HPW))

    def load_half(t, h, sem):
        (g, l) = halves_of(t)[h]
        return pltpu.async_copy(x_hbm.at[pl.ds(g, HPW)],
                                slab.at[pl.ds(l, HPW)], sem)

    def store_half(t, h, sem):
        (g, l) = halves_of(t)[h]
        return pltpu.async_copy(slab.at[pl.ds(l, HPW)],
                                out_hbm.at[pl.ds(g, HPW)], sem)

    def partial_store(t):
        base_row = base_of(t)
        rows = jnp.clip(M - base_row, 0, S)
        my_b0 = s * BPW
        my_n = jnp.clip(rows // RB - my_b0, 0, BPW)

        @pl.when(my_n == BPW)
        def _full():
            o = my_b0 * RB
            pltpu.sync_copy(slab.at[pl.ds(o, SPW)],
                            out_hbm.at[pl.ds(base_row + o, SPW)])

        @pl.when(jnp.logical_and(my_n > 0, my_n < BPW))
        def _part():
            def blk(j, _):
                o = (my_b0 + j) * RB
                pltpu.sync_copy(slab.at[pl.ds(o, RB)],
                                out_hbm.at[pl.ds(base_row + o, RB)])
                return 0
            lax.fori_loop(0, my_n, blk, 0)

    def _load(j):
        return pltpu.async_copy(
            src_hbm.at[pl.ds(s * EPW + j * CH, CH)],
            bufs[j % NBUF], lsems[j % NBUF])

    ld = (load_half(0, 0, hsem0), load_half(0, 1, hsem1))
    loads = {0: _load(0), 1: _load(1)}
    pltpu.sync_copy(idx_hbm.at[pl.ds(s * EPW, EPW)], idx_v)
    pltpu.sync_copy(alpha_hbm, alpha_v)
    av = alpha_v[pl.ds(0, L)]
    do_scale = av[0] != 1.0

    for t in range(SWEEPS):
        base_row = base_of(t)
        rows = jnp.clip(M - base_row, 0, S)

        lo = base_row
        hi = base_row + rows
        for r in range(NCHUNK):
            for q in range(NV):
                vec = idx_v[pl.ds((r * NV + q) * L, L)]
                in_slab = (vec >= lo) & (vec < hi)
                tgt_v[r, pl.ds(q * L, L)] = jnp.where(in_slab, vec - lo, garb)

        if ld is not None:
            ld[0].wait()
            ld[1].wait()
            ld = None
        plsc.subcore_barrier()

        adds = {}
        for j in range(NCHUNK):
            loads[j].wait()
            bj = bufs[j % NBUF]

            @pl.when(do_scale)
            def _scale(bj=bj):
                def scale_row(rr, _):
                    for q in range(D // L):
                        sl = pl.ds(q * L, L)
                        bj[rr, sl] = bj[rr, sl] * av
                    return 0
                lax.fori_loop(0, CH, scale_row, 0)

            adds[j] = pltpu.async_copy(
                bj, slab.at[tgt_v.at[j]], asems[j % NBUF], add=True)
            if j + 2 < NCHUNK:
                if j - (NBUF - 2) >= 0:
                    adds[j - (NBUF - 2)].wait()
                loads[j + 2] = _load(j + 2)
        for j in range(max(0, NCHUNK - NBUF), NCHUNK):
            adds[j].wait()
        if t + 1 < SWEEPS:
            loads = {0: _load(0), 1: _load(1)}

        plsc.subcore_barrier()

        if t < FULL:
            st = (store_half(t, 0, ssem0), store_half(t, 1, ssem1))
            st[0].wait()
            nl0 = load_half(t + 1, 0, hsem0)
            st[1].wait()
            nl1 = load_half(t + 1, 1, hsem1)
            ld = (nl0, nl1)
        else:
            partial_store(t)


_scatter_add = functools.partial(
    pl.kernel,
    mesh=plsc.VectorSubcoreMesh(core_axis_name="c", subcore_axis_name="s"),
    out_type=jax.ShapeDtypeStruct((M, D), jnp.float32),
    scratch_types=(
        [
            pltpu.VMEM((EPW,), jnp.int32),
            pltpu.VMEM((NCHUNK, CH), jnp.int32),
            pltpu.VMEM((L,), jnp.float32),
            pltpu.VMEM_SHARED((S + NS * GR, D), jnp.float32),
        ]
        + [pltpu.VMEM((CH, D), jnp.float32) for _ in range(NBUF)]
        + [pltpu.SemaphoreType.DMA for _ in range(2 * NBUF + 4)]
    ),
)(_body)


def kernel(x, dim, index, source, alpha):
    del dim
    alpha_vec = jnp.full((L,), alpha, dtype=jnp.float32)
    return _scatter_add(x, index, source, alpha_vec)
